# baseline (speedup 1.0000x reference)
import functools

import jax
import jax.numpy as jnp
from jax import lax
from jax.experimental import pallas as pl
from jax.experimental.pallas import tpu as pltpu
from jax.experimental.pallas import tpu_sc as plsc

_BATCH = 16384
_D = 32
_NCLASS = 100000
_NC = 2
_NS = 16
_L = 16
_NW = _NC * _NS
_BPW = _BATCH // _NW
_BLKS = _BPW // _L

_mesh = plsc.VectorSubcoreMesh(core_axis_name="c", subcore_axis_name="s")


@functools.partial(
    pl.kernel,
    out_type=jax.ShapeDtypeStruct((_NW * _L,), jnp.float32),
    mesh=_mesh,
    compiler_params=pltpu.CompilerParams(
        needs_layout_passes=False, use_tc_tiling_on_sc=False),
    scratch_types=[
        pltpu.VMEM((_BPW,), jnp.int32),
        pltpu.VMEM((_BPW * _D,), jnp.float32),
        pltpu.VMEM((_BPW, _D), jnp.float32),
        pltpu.VMEM((_L * _L,), jnp.float32),
        pltpu.VMEM((_L,), jnp.float32),
        pltpu.SemaphoreType.DMA,
    ],
)
def _center_loss_sc(x_hbm, labels_hbm, centers_hbm, out_hbm,
                    idx_v, xv, cv, tmp, accv, sem):
    wid = lax.axis_index("s") * _NC + lax.axis_index("c")
    base = wid * _BPW

    pltpu.sync_copy(labels_hbm.at[pl.ds(base, _BPW)], idx_v)
    gather = pltpu.async_copy(centers_hbm.at[idx_v], cv, sem)
    pltpu.sync_copy(x_hbm.at[pl.ds(base * _D, _BPW * _D)], xv)
    gather.wait()

    lanes = lax.iota(jnp.int32, _L)

    def block(blk, acc):
        row0 = blk * _L
        for j in range(_L):
            d1 = xv[pl.ds((row0 + j) * _D, _L)] - cv[row0 + j, pl.ds(0, _L)]
            d2 = (xv[pl.ds((row0 + j) * _D + _L, _L)]
                  - cv[row0 + j, pl.ds(_L, _L)])
            tmp[pl.ds(j * _L, _L)] = d1 * d1 + d2 * d2
        dacc = jnp.zeros((_L,), jnp.float32)
        base_idx = lanes * _L
        for k in range(_L):
            dacc = dacc + plsc.load_gather(tmp, [base_idx + k])
        dist = jnp.clip(dacc, 1e-12, 1e12)
        return acc + dist

    acc = lax.fori_loop(0, _BLKS, block, jnp.zeros((_L,), jnp.float32))
    accv[...] = acc
    pltpu.sync_copy(accv, out_hbm.at[pl.ds(wid * _L, _L)])


def kernel(x, labels, centers):
    partials = _center_loss_sc(
        x.reshape(-1), labels.astype(jnp.int32), centers)
    return jnp.sum(partials) * (1.0 / _BATCH)


# speedup vs baseline: 1.0001x; 1.0001x over previous
import functools

import jax
import jax.numpy as jnp
from jax import lax
from jax.experimental import pallas as pl
from jax.experimental.pallas import tpu as pltpu
from jax.experimental.pallas import tpu_sc as plsc

_BATCH = 16384
_D = 32
_NCLASS = 100000
_NC = 2
_NS = 16
_L = 16
_NW = _NC * _NS
_BPW = _BATCH // _NW
_BLKS = _BPW // _L

_mesh = plsc.VectorSubcoreMesh(core_axis_name="c", subcore_axis_name="s")


@functools.partial(
    pl.kernel,
    out_type=jax.ShapeDtypeStruct((_NW * _L,), jnp.float32),
    mesh=_mesh,
    compiler_params=pltpu.CompilerParams(
        needs_layout_passes=False, use_tc_tiling_on_sc=False),
    scratch_types=[
        pltpu.VMEM((_BPW,), jnp.int32),
        pltpu.VMEM((_BPW, _D), jnp.float32),
        pltpu.VMEM((_BPW, _D), jnp.float32),
        pltpu.VMEM((_L * _L,), jnp.float32),
        pltpu.VMEM((_L,), jnp.float32),
        pltpu.SemaphoreType.DMA,
    ],
)
def _center_loss_sc(x_hbm, labels_hbm, centers_hbm, out_hbm,
                    idx_v, xv, cv, tmp, accv, sem):
    wid = lax.axis_index("s") * _NC + lax.axis_index("c")
    base = wid * _BPW

    pltpu.sync_copy(labels_hbm.at[pl.ds(base, _BPW)], idx_v)
    gather = pltpu.async_copy(centers_hbm.at[idx_v], cv, sem)
    pltpu.sync_copy(x_hbm.at[pl.ds(base, _BPW)], xv)
    gather.wait()

    lanes = lax.iota(jnp.int32, _L)

    def block(blk, acc):
        row0 = blk * _L
        for j in range(_L):
            d1 = xv[row0 + j, pl.ds(0, _L)] - cv[row0 + j, pl.ds(0, _L)]
            d2 = xv[row0 + j, pl.ds(_L, _L)] - cv[row0 + j, pl.ds(_L, _L)]
            tmp[pl.ds(j * _L, _L)] = d1 * d1 + d2 * d2
        dacc = jnp.zeros((_L,), jnp.float32)
        base_idx = lanes * _L
        for k in range(_L):
            dacc = dacc + plsc.load_gather(tmp, [base_idx + k])
        dist = jnp.clip(dacc, 1e-12, 1e12)
        return acc + dist

    acc = lax.fori_loop(0, _BLKS, block, jnp.zeros((_L,), jnp.float32))
    accv[...] = acc
    pltpu.sync_copy(accv, out_hbm.at[pl.ds(wid * _L, _L)])


def kernel(x, labels, centers):
    partials = _center_loss_sc(x, labels.astype(jnp.int32), centers)
    return jnp.sum(partials) * (1.0 / _BATCH)


# speedup vs baseline: 1.1543x; 1.1542x over previous
import functools

import jax
import jax.numpy as jnp
from jax import lax
from jax.experimental import pallas as pl
from jax.experimental.pallas import tpu as pltpu
from jax.experimental.pallas import tpu_sc as plsc

_BATCH = 16384
_D = 32
_NCLASS = 100000
_NC = 2
_NS = 16
_L = 16
_NW = _NC * _NS
_BPW = _BATCH // _NW
_CH = 128

_mesh = plsc.VectorSubcoreMesh(core_axis_name="c", subcore_axis_name="s")


@functools.partial(
    pl.kernel,
    out_type=jax.ShapeDtypeStruct((_NW * _L,), jnp.float32),
    mesh=_mesh,
    compiler_params=pltpu.CompilerParams(
        needs_layout_passes=False, use_tc_tiling_on_sc=True),
    scratch_types=[
        pltpu.VMEM((_BPW,), jnp.int32),
        pltpu.VMEM((_CH, _D), jnp.float32),
        pltpu.VMEM((_CH, _D), jnp.float32),
        pltpu.VMEM((_L * _L,), jnp.float32),
        pltpu.VMEM((_L,), jnp.float32),
        pltpu.SemaphoreType.DMA,
        pltpu.SemaphoreType.DMA,
    ],
)
def _center_loss_sc(x_hbm, labels_hbm, centers_hbm, out_hbm,
                    idx_v, xv, cv, tmp, accv, semg, semx):
    wid = lax.axis_index("s") * _NC + lax.axis_index("c")
    base = wid * _BPW

    pltpu.sync_copy(labels_hbm.at[pl.ds(base, _BPW)], idx_v)

    lanes = lax.iota(jnp.int32, _L)

    def issue(g, carry):
        vec = idx_v[pl.ds(g * _L, _L)]
        for j in range(_L):
            r = vec[j]
            pltpu.async_copy(centers_hbm.at[pl.ds(r, 1)],
                             cv.at[pl.ds(g * _L + j - carry, 1)], semg)
        return carry

    def block(blk, acc):
        row0 = blk * _L
        for j in range(_L):
            d1 = xv[row0 + j, pl.ds(0, _L)] - cv[row0 + j, pl.ds(0, _L)]
            d2 = xv[row0 + j, pl.ds(_L, _L)] - cv[row0 + j, pl.ds(_L, _L)]
            tmp[pl.ds(j * _L, _L)] = d1 * d1 + d2 * d2
        dacc = jnp.zeros((_L,), jnp.float32)
        base_idx = lanes * _L
        for k in range(_L):
            dacc = dacc + plsc.load_gather(tmp, [base_idx + k])
        dist = jnp.clip(dacc, 1e-12, 1e12)
        return acc + dist

    acc = jnp.zeros((_L,), jnp.float32)
    for c in range(0, _BPW, _CH):
        xcp = pltpu.async_copy(x_hbm.at[pl.ds(base + c, _CH)], xv, semx)
        lax.fori_loop(c // _L, (c + _CH) // _L, issue, c)
        pltpu.make_async_copy(centers_hbm.at[pl.ds(0, _CH)], cv, semg).wait()
        xcp.wait()
        acc = lax.fori_loop(0, _CH // _L, block, acc)

    accv[...] = acc
    pltpu.sync_copy(accv, out_hbm.at[pl.ds(wid * _L, _L)])


def kernel(x, labels, centers):
    partials = _center_loss_sc(x, labels.astype(jnp.int32), centers)
    return jnp.sum(partials) * (1.0 / _BATCH)
